# baseline (speedup 1.0000x reference)
import jax
import jax.numpy as jnp
from jax import lax
from jax.experimental import pallas as pl
from jax.experimental.pallas import tpu as pltpu
from jax.experimental.pallas import tpu_sc as plsc

N_ATOMS = 16384
N_GRAPHS = 16
TOP_B = 5
NS = 16
CHUNK = N_ATOMS // NS
LANES = 16


def _sc_body(batch_hbm, aw_hbm, si_hbm, sw_hbm,
             batch_v, hist_v, allhist_v, inv_v, out_v, sel_i_v, sel_w_v,
             shared_hist):
    s = lax.axis_index("s")
    base = s * CHUNK

    pltpu.sync_copy(batch_hbm.at[pl.ds(base, CHUNK)], batch_v)

    lane16 = lax.iota(jnp.int32, LANES) * LANES
    zeros = jnp.zeros((LANES,), jnp.int32)
    ones = jnp.ones((LANES,), jnp.int32)

    for l in range(LANES):
        hist_v[pl.ds(l * LANES, LANES)] = zeros
    for i in range(CHUNK // LANES):
        idx = batch_v[pl.ds(i * LANES, LANES)]
        plsc.addupdate_scatter(hist_v, [lane16 + idx], ones)
    counts_l = hist_v[pl.ds(0, LANES)]
    for l in range(1, LANES):
        counts_l = counts_l + hist_v[pl.ds(l * LANES, LANES)]

    hist_v[pl.ds(0, LANES)] = counts_l
    pltpu.sync_copy(hist_v.at[pl.ds(0, LANES)],
                    shared_hist.at[pl.ds(s * LANES, LANES)])
    plsc.subcore_barrier()
    pltpu.sync_copy(shared_hist, allhist_v)

    counts = allhist_v[pl.ds(0, LANES)]
    for i in range(1, NS):
        counts = counts + allhist_v[pl.ds(i * LANES, LANES)]
    inv = 1.0 / counts.astype(jnp.float32)
    for l in range(LANES):
        inv_v[pl.ds(l * LANES, LANES)] = inv

    for i in range(CHUNK // LANES):
        idx = batch_v[pl.ds(i * LANES, LANES)]
        out_v[pl.ds(i * LANES, LANES)] = plsc.load_gather(
            inv_v, [lane16 + idx])
    pltpu.sync_copy(out_v, aw_hbm.at[pl.ds(base, CHUNK)])

    @pl.when(s == 0)
    def _():
        starts = plsc.cumsum(counts) - counts
        gid = lax.iota(jnp.int32, LANES)
        for j in range(TOP_B):
            pos = gid * TOP_B + j
            plsc.store_scatter(sel_i_v, [pos], starts + j)
            plsc.store_scatter(sel_w_v, [pos], inv)
        pltpu.sync_copy(sel_i_v, si_hbm)
        pltpu.sync_copy(sel_w_v, sw_hbm)


_sc_call = pl.kernel(
    _sc_body,
    out_type=(
        jax.ShapeDtypeStruct((N_ATOMS,), jnp.float32),
        jax.ShapeDtypeStruct((N_GRAPHS * TOP_B,), jnp.int32),
        jax.ShapeDtypeStruct((N_GRAPHS * TOP_B,), jnp.float32),
    ),
    mesh=plsc.VectorSubcoreMesh(
        core_axis_name="c", subcore_axis_name="s", num_cores=1,
        num_subcores=NS),
    compiler_params=pltpu.CompilerParams(needs_layout_passes=False),
    scratch_types=(
        pltpu.VMEM((CHUNK,), jnp.int32),
        pltpu.VMEM((LANES * LANES,), jnp.int32),
        pltpu.VMEM((NS * LANES,), jnp.int32),
        pltpu.VMEM((LANES * LANES,), jnp.float32),
        pltpu.VMEM((CHUNK,), jnp.float32),
        pltpu.VMEM((N_GRAPHS * TOP_B,), jnp.int32),
        pltpu.VMEM((N_GRAPHS * TOP_B,), jnp.float32),
        pltpu.VMEM_SHARED((NS * LANES,), jnp.int32),
    ),
)


@jax.jit
def kernel(node_repr, graph_repr, prototypes, batch):
    atom_weights, selected_indices, selected_weights = _sc_call(
        batch.astype(jnp.int32))
    return (
        graph_repr,
        atom_weights,
        selected_indices.astype(batch.dtype),
        selected_weights,
    )


# speedup vs baseline: 5.2050x; 5.2050x over previous
import jax
import jax.numpy as jnp
from jax import lax
from jax.experimental import pallas as pl

N_ATOMS = 16384
N_GRAPHS = 16
TOP_B = 5
ROWS = 128
COLS = 128


def _tc_body(b_ref, gr_ref, gro_ref, aw_ref, si_ref, sw_ref):
    b = b_ref[...]

    gro_ref[...] = gr_ref[...]

    counts = [jnp.sum((b == g).astype(jnp.int32)) for g in range(N_GRAPHS)]
    inv = [1.0 / c.astype(jnp.float32) for c in counts]

    aw = jnp.full((ROWS, COLS), inv[0], jnp.float32)
    for g in range(1, N_GRAPHS):
        aw = jnp.where(b == g, inv[g], aw)
    aw_ref[...] = aw

    starts = []
    acc = jnp.int32(0)
    for g in range(N_GRAPHS):
        starts.append(acc)
        acc = acc + counts[g]

    gmap = lax.broadcasted_iota(jnp.int32, (N_GRAPHS, TOP_B), 0)
    jmap = lax.broadcasted_iota(jnp.int32, (N_GRAPHS, TOP_B), 1)
    si = jnp.full((N_GRAPHS, TOP_B), starts[0], jnp.int32)
    sw = jnp.full((N_GRAPHS, TOP_B), inv[0], jnp.float32)
    for g in range(1, N_GRAPHS):
        si = jnp.where(gmap == g, starts[g], si)
        sw = jnp.where(gmap == g, inv[g], sw)
    si_ref[...] = si + jmap
    sw_ref[...] = sw


_tc_call = pl.pallas_call(
    _tc_body,
    out_shape=(
        jax.ShapeDtypeStruct((N_GRAPHS, 512), jnp.float32),
        jax.ShapeDtypeStruct((ROWS, COLS), jnp.float32),
        jax.ShapeDtypeStruct((N_GRAPHS, TOP_B), jnp.int32),
        jax.ShapeDtypeStruct((N_GRAPHS, TOP_B), jnp.float32),
    ),
)


@jax.jit
def kernel(node_repr, graph_repr, prototypes, batch):
    gro, aw2, si2, sw2 = _tc_call(
        batch.astype(jnp.int32).reshape(ROWS, COLS), graph_repr)
    return (
        gro,
        aw2.reshape(N_ATOMS),
        si2.reshape(N_GRAPHS * TOP_B).astype(batch.dtype),
        sw2.reshape(N_GRAPHS * TOP_B),
    )
